# baseline (speedup 1.0000x reference)
import jax
import jax.numpy as jnp
from jax import lax
from jax.experimental import pallas as pl
from jax.experimental.pallas import tpu as pltpu
from jax.experimental.pallas import tpu_sc as plsc

NUM_UNITS = 1000000
NUM_PHONEMES = 32
BATCH = 16384
HIST = 200

NW = 32
LANES = 128
BCHUNKS = BATCH // LANES
CHUNKS = HIST * BCHUNKS
CHUNKS_PER_W = CHUNKS // NW
DTILES = NUM_PHONEMES // 8


def _gather_kernel(xt_hbm, wd_hbm, out_hbm, idxb, qb, cb, rows, tb,
                   gsem0, gsem1, ssem0, ssem1):
    wid = lax.axis_index("s") * 2 + lax.axis_index("c")
    c0 = wid * CHUNKS_PER_W
    gsem = (gsem0, gsem1)
    ssem = (ssem0, ssem1)

    def coords(i):
        ch = c0 + i
        return ch // BCHUNKS, ch % BCHUNKS

    def load_and_fire(p, i):
        h, tc = coords(i)
        pltpu.sync_copy(xt_hbm.at[h, pl.ds(tc * LANES, LANES)], idxb.at[p])
        for v in range(8):
            iv = idxb[p, pl.ds(16 * v, 16)]
            qb[p, pl.ds(16 * v, 16)] = lax.shift_right_logical(iv, 2)
            cb[p, pl.ds(16 * v, 16)] = lax.shift_left(
                lax.bitwise_and(iv, 3), 5)
        pltpu.async_copy(wd_hbm.at[qb.at[p]], rows.at[p], gsem[p])

    def drain_gather(p):
        pltpu.make_async_copy(wd_hbm.at[qb.at[p]], rows.at[p],
                              gsem[p]).wait()

    def transpose_chunk(p):
        for v in range(8):
            rowv = lax.iota(jnp.int32, 16) + 16 * v
            cbv = cb[p, pl.ds(16 * v, 16)]
            for d in range(NUM_PHONEMES):
                val = plsc.load_gather(rows.at[p], [rowv, cbv + d])
                tb[p, d // 8, d % 8, pl.ds(16 * v, 16)] = val

    def fire_stores(p, i):
        h, tc = coords(i)
        for tr in range(DTILES):
            pltpu.async_copy(tb.at[p, tr], out_hbm.at[h, tr * BCHUNKS + tc],
                             ssem[p])

    def drain_stores(p, i):
        h, tc = coords(i)
        for tr in range(DTILES):
            pltpu.make_async_copy(tb.at[p, tr],
                                  out_hbm.at[h, tr * BCHUNKS + tc],
                                  ssem[p]).wait()

    def body(i, p, drain_prev_store, process_prev):
        q = 1 - p
        if drain_prev_store:
            drain_stores(p, i - 2)
        load_and_fire(p, i)
        if process_prev:
            drain_gather(q)
            transpose_chunk(q)
            fire_stores(q, i - 1)

    body(0, 0, False, False)
    body(1, 1, False, True)

    def loop_body(k, _):
        i = 2 * k
        body(i, 0, True, True)
        body(i + 1, 1, True, True)
        return _

    lax.fori_loop(1, CHUNKS_PER_W // 2, loop_body, None)

    last = CHUNKS_PER_W - 1
    drain_gather(1)
    transpose_chunk(1)
    fire_stores(1, last)
    drain_stores(0, last - 1)
    drain_stores(1, last)


@jax.jit
def _run(xt, wd):
    mesh = plsc.VectorSubcoreMesh(core_axis_name="c", subcore_axis_name="s")
    out4 = pl.kernel(
        _gather_kernel,
        out_type=jax.ShapeDtypeStruct((HIST, DTILES * BCHUNKS, 8, LANES),
                                      jnp.float32),
        mesh=mesh,
        scratch_types=[
            pltpu.VMEM((2, LANES), jnp.int32),
            pltpu.VMEM((2, LANES), jnp.int32),
            pltpu.VMEM((2, LANES), jnp.int32),
            pltpu.VMEM((2, LANES, LANES), jnp.float32),
            pltpu.VMEM((2, DTILES, 8, LANES), jnp.float32),
            pltpu.SemaphoreType.DMA,
            pltpu.SemaphoreType.DMA,
            pltpu.SemaphoreType.DMA,
            pltpu.SemaphoreType.DMA,
        ],
        compiler_params=pltpu.CompilerParams(needs_layout_passes=False),
    )(xt, wd)
    out5 = out4.reshape(HIST, DTILES, BCHUNKS, 8, LANES)
    return out5.transpose(2, 4, 0, 1, 3).reshape(BATCH, HIST, NUM_PHONEMES)


def kernel(x, W):
    xt = x.astype(jnp.int32).T
    wd = W.reshape(NUM_UNITS // 4, 4 * NUM_PHONEMES)
    return _run(xt, wd)


# speedup vs baseline: 1.4916x; 1.4916x over previous
import jax
import jax.numpy as jnp
from jax import lax
from jax.experimental import pallas as pl
from jax.experimental.pallas import tpu as pltpu
from jax.experimental.pallas import tpu_sc as plsc

NUM_UNITS = 1000000
NUM_PHONEMES = 32
BATCH = 16384
HIST = 200

NW = 32
ROWS_PER_W = BATCH // NW
XROWS_PER_GROUP = 4
IDX_PER_GROUP = XROWS_PER_GROUP * HIST
GROUPS_PER_W = ROWS_PER_W // XROWS_PER_GROUP
GATHER_SPLITS = [(0, 128), (128, 128), (256, 128), (384, 128), (512, 128),
                 (640, 128), (768, 32)]
TOTAL = BATCH * HIST


def _gather_kernel(x_hbm, w_hbm, out_hbm, idx_buf, rows, gsem0, gsem1,
                   ssem0, ssem1):
    wid = lax.axis_index("s") * 2 + lax.axis_index("c")
    b0 = wid * ROWS_PER_W
    gsem = (gsem0, gsem1)
    ssem = (ssem0, ssem1)

    def fire_gathers(p):
        for (off, n) in GATHER_SPLITS:
            pltpu.async_copy(w_hbm.at[idx_buf.at[p, pl.ds(off, n)]],
                             rows.at[p, pl.ds(off, n)], gsem[p])

    def drain_gathers(p):
        for (off, n) in GATHER_SPLITS:
            pltpu.make_async_copy(w_hbm.at[idx_buf.at[p, pl.ds(off, n)]],
                                  rows.at[p, pl.ds(off, n)], gsem[p]).wait()

    def fire_stores(p, u):
        for i in range(XROWS_PER_GROUP):
            pltpu.async_copy(rows.at[p, pl.ds(i * HIST, HIST)],
                             out_hbm.at[u * XROWS_PER_GROUP + i], ssem[p])

    def drain_stores(p, u):
        for i in range(XROWS_PER_GROUP):
            pltpu.make_async_copy(rows.at[p, pl.ds(i * HIST, HIST)],
                                  out_hbm.at[u * XROWS_PER_GROUP + i],
                                  ssem[p]).wait()

    def load_idx(p, u):
        pltpu.sync_copy(x_hbm.at[pl.ds(u * IDX_PER_GROUP, IDX_PER_GROUP)],
                        idx_buf.at[p])

    def body(u, p, drain_prev_store, process_prev):
        q = 1 - p
        if drain_prev_store:
            drain_stores(p, u - 2)
        load_idx(p, u)
        fire_gathers(p)
        if process_prev:
            drain_gathers(q)
            fire_stores(q, u - 1)

    u0 = wid * GROUPS_PER_W
    body(u0, 0, False, False)
    body(u0 + 1, 1, False, True)

    def loop_body(k, _):
        u = u0 + 2 * k
        body(u, 0, True, True)
        body(u + 1, 1, True, True)
        return _

    lax.fori_loop(1, GROUPS_PER_W // 2, loop_body, None)

    last = u0 + GROUPS_PER_W - 1
    drain_gathers(1)
    fire_stores(1, last)
    drain_stores(0, last - 1)
    drain_stores(1, last)


@jax.jit
def _run(x_flat, w):
    mesh = plsc.VectorSubcoreMesh(core_axis_name="c", subcore_axis_name="s")
    return pl.kernel(
        _gather_kernel,
        out_type=jax.ShapeDtypeStruct((BATCH, HIST, NUM_PHONEMES),
                                      jnp.float32),
        mesh=mesh,
        scratch_types=[
            pltpu.VMEM((2, IDX_PER_GROUP), jnp.int32),
            pltpu.VMEM((2, IDX_PER_GROUP, NUM_PHONEMES), jnp.float32),
            pltpu.SemaphoreType.DMA,
            pltpu.SemaphoreType.DMA,
            pltpu.SemaphoreType.DMA,
            pltpu.SemaphoreType.DMA,
        ],
        compiler_params=pltpu.CompilerParams(use_tc_tiling_on_sc=False),
    )(x_flat, w)


def kernel(x, W):
    x_flat = x.astype(jnp.int32).reshape(TOTAL)
    return _run(x_flat, W)


# speedup vs baseline: 2.4001x; 1.6090x over previous
import jax
import jax.numpy as jnp
from jax import lax
from jax.experimental import pallas as pl
from jax.experimental.pallas import tpu as pltpu
from jax.experimental.pallas import tpu_sc as plsc

NUM_UNITS = 1000000
NUM_PHONEMES = 32
BATCH = 16384
HIST = 200
HPAD = 208

NW = 32
LANES = 128
BCHUNKS = BATCH // LANES
TC_PER_W = BCHUNKS // NW
DTILES = NUM_PHONEMES // 8


def _gather_kernel(x_hbm, w_hbm, out_hbm, xblock, xblockt, rows, tb,
                   gsem0, gsem1, ssem0, ssem1):
    wid = lax.axis_index("s") * 2 + lax.axis_index("c")
    gsem = (gsem0, gsem1)
    ssem = (ssem0, ssem1)
    jv = lax.iota(jnp.int32, 16)
    bvs = [jv + 16 * v for v in range(8)]

    def transpose_xblock():
        def txb(k, carry):
            dlt = lax.bitwise_and(jv + k, 15)
            for h16 in range(HPAD // 16):
                hj = dlt + 16 * h16
                for v in range(8):
                    val = plsc.load_gather(xblock, [bvs[v], hj])
                    plsc.store_scatter(xblockt, [hj, bvs[v]], val)
            return carry
        lax.fori_loop(0, 16, txb, None)

    def fire_gather(p, h):
        pltpu.async_copy(w_hbm.at[xblockt.at[h]], rows.at[p], gsem[p])

    def drain_gather(p, h):
        pltpu.make_async_copy(w_hbm.at[xblockt.at[h]], rows.at[p],
                              gsem[p]).wait()

    def transpose_chunk(p):
        def tck(k, carry):
            dj = lax.bitwise_and(jv + k, 31)
            dt = lax.shift_right_logical(dj, 3)
            dsub = lax.bitwise_and(dj, 7)
            for v in range(8):
                val = plsc.load_gather(rows.at[p], [bvs[v], dj])
                plsc.store_scatter(tb.at[p], [dt, dsub, bvs[v]], val)
            return carry
        lax.fori_loop(0, NUM_PHONEMES, tck, None)

    def fire_stores(p, h, tc):
        for tr in range(DTILES):
            pltpu.async_copy(tb.at[p, tr], out_hbm.at[h, tr * BCHUNKS + tc],
                             ssem[p])

    def drain_stores(p, h, tc):
        for tr in range(DTILES):
            pltpu.make_async_copy(tb.at[p, tr],
                                  out_hbm.at[h, tr * BCHUNKS + tc],
                                  ssem[p]).wait()

    def do_block(tc_local, carry):
        tc = wid * TC_PER_W + tc_local
        pltpu.sync_copy(x_hbm.at[pl.ds(tc * LANES, LANES), :],
                        xblock.at[:, pl.ds(0, HIST)])
        transpose_xblock()

        def body(h, p, drain_prev_store, process_prev):
            q = 1 - p
            if drain_prev_store:
                drain_stores(p, h - 2, tc)
            fire_gather(p, h)
            if process_prev:
                drain_gather(q, h - 1)
                transpose_chunk(q)
                fire_stores(q, h - 1, tc)

        body(0, 0, False, False)
        body(1, 1, False, True)

        def loop_body(k, c):
            h = 2 * k
            body(h, 0, True, True)
            body(h + 1, 1, True, True)
            return c

        lax.fori_loop(1, HIST // 2, loop_body, None)

        drain_gather(1, HIST - 1)
        transpose_chunk(1)
        fire_stores(1, HIST - 1, tc)
        drain_stores(0, HIST - 2, tc)
        drain_stores(1, HIST - 1, tc)
        return carry

    lax.fori_loop(0, TC_PER_W, do_block, None)


@jax.jit
def _run(x, w):
    mesh = plsc.VectorSubcoreMesh(core_axis_name="c", subcore_axis_name="s")
    out4 = pl.kernel(
        _gather_kernel,
        out_type=jax.ShapeDtypeStruct((HIST, DTILES * BCHUNKS, 8, LANES),
                                      jnp.float32),
        mesh=mesh,
        scratch_types=[
            pltpu.VMEM((LANES, HPAD), jnp.int32),
            pltpu.VMEM((HPAD, LANES), jnp.int32),
            pltpu.VMEM((2, LANES, NUM_PHONEMES), jnp.float32),
            pltpu.VMEM((2, DTILES, 8, LANES), jnp.float32),
            pltpu.SemaphoreType.DMA,
            pltpu.SemaphoreType.DMA,
            pltpu.SemaphoreType.DMA,
            pltpu.SemaphoreType.DMA,
        ],
        compiler_params=pltpu.CompilerParams(use_tc_tiling_on_sc=False,
                                             needs_layout_passes=False),
    )(x, w)
    out5 = out4.reshape(HIST, DTILES, BCHUNKS, 8, LANES)
    return out5.transpose(2, 4, 0, 1, 3).reshape(BATCH, HIST, NUM_PHONEMES)


def kernel(x, W):
    return _run(x.astype(jnp.int32), W)


# speedup vs baseline: 3.2608x; 1.3586x over previous
import jax
import jax.numpy as jnp
from jax import lax
from jax.experimental import pallas as pl
from jax.experimental.pallas import tpu as pltpu
from jax.experimental.pallas import tpu_sc as plsc

NUM_UNITS = 1000000
NUM_PHONEMES = 32
BATCH = 16384
HIST = 200
HPAD = 208

NW = 32
LANES = 128
BCHUNKS = BATCH // LANES
TC_PER_W = BCHUNKS // NW
DTILES = NUM_PHONEMES // 8


def _gather_kernel(x_hbm, w_hbm, out_hbm, xblock, xblockt, rows, tb,
                   gsem0, gsem1, ssem0, ssem1):
    wid = lax.axis_index("s") * 2 + lax.axis_index("c")
    gsem = (gsem0, gsem1)
    ssem = (ssem0, ssem1)
    jv = lax.iota(jnp.int32, 16)
    bvs = [jv + 16 * v for v in range(8)]

    def transpose_xblock():
        def txb(k, carry):
            dlt = lax.bitwise_and(jv + k, 15)
            for h16 in range(HPAD // 16):
                hj = dlt + 16 * h16
                for v in range(8):
                    val = plsc.load_gather(xblock, [bvs[v], hj])
                    plsc.store_scatter(xblockt, [hj, bvs[v]], val)
            return carry
        lax.fori_loop(0, 16, txb, None)

    def fire_gather(p, h):
        pltpu.async_copy(w_hbm.at[xblockt.at[h]], rows.at[p], gsem[p])

    def drain_gather(p, h):
        pltpu.make_async_copy(w_hbm.at[xblockt.at[h]], rows.at[p],
                              gsem[p]).wait()

    def transpose_chunk(p):
        def tck(k2, carry):
            for u in range(2):
                k = 2 * k2 + u
                dj = lax.bitwise_and(jv + k, 31)
                dt = lax.shift_right_logical(dj, 3)
                dsub = lax.bitwise_and(dj, 7)
                vals = [plsc.load_gather(rows.at[p], [bvs[v], dj])
                        for v in range(8)]
                for v in range(8):
                    plsc.store_scatter(tb.at[p], [dt, dsub, bvs[v]], vals[v])
            return carry
        lax.fori_loop(0, NUM_PHONEMES // 2, tck, None)

    def fire_stores(p, h, tc):
        for tr in range(DTILES):
            pltpu.async_copy(tb.at[p, tr], out_hbm.at[h, tr * BCHUNKS + tc],
                             ssem[p])

    def drain_stores(p, h, tc):
        for tr in range(DTILES):
            pltpu.make_async_copy(tb.at[p, tr],
                                  out_hbm.at[h, tr * BCHUNKS + tc],
                                  ssem[p]).wait()

    def do_block(tc_local, carry):
        tc = wid * TC_PER_W + tc_local
        pltpu.sync_copy(x_hbm.at[pl.ds(tc * LANES, LANES), :],
                        xblock.at[:, pl.ds(0, HIST)])
        transpose_xblock()

        def body(h, p, drain_prev_store, process_prev):
            q = 1 - p
            if drain_prev_store:
                drain_stores(p, h - 2, tc)
            fire_gather(p, h)
            if process_prev:
                drain_gather(q, h - 1)
                transpose_chunk(q)
                fire_stores(q, h - 1, tc)

        body(0, 0, False, False)
        body(1, 1, False, True)

        def loop_body(k, c):
            h = 2 * k
            body(h, 0, True, True)
            body(h + 1, 1, True, True)
            return c

        lax.fori_loop(1, HIST // 2, loop_body, None)

        drain_gather(1, HIST - 1)
        transpose_chunk(1)
        fire_stores(1, HIST - 1, tc)
        drain_stores(0, HIST - 2, tc)
        drain_stores(1, HIST - 1, tc)
        return carry

    lax.fori_loop(0, TC_PER_W, do_block, None)


@jax.jit
def _run(x, w):
    mesh = plsc.VectorSubcoreMesh(core_axis_name="c", subcore_axis_name="s")
    out4 = pl.kernel(
        _gather_kernel,
        out_type=jax.ShapeDtypeStruct((HIST, DTILES * BCHUNKS, 8, LANES),
                                      jnp.float32),
        mesh=mesh,
        scratch_types=[
            pltpu.VMEM((LANES, HPAD), jnp.int32),
            pltpu.VMEM((HPAD, LANES), jnp.int32),
            pltpu.VMEM((2, LANES, NUM_PHONEMES), jnp.float32),
            pltpu.VMEM((2, DTILES, 8, LANES), jnp.float32),
            pltpu.SemaphoreType.DMA,
            pltpu.SemaphoreType.DMA,
            pltpu.SemaphoreType.DMA,
            pltpu.SemaphoreType.DMA,
        ],
        compiler_params=pltpu.CompilerParams(use_tc_tiling_on_sc=False,
                                             needs_layout_passes=False),
    )(x, w)
    out5 = out4.reshape(HIST, DTILES, BCHUNKS, 8, LANES)
    return out5.transpose(2, 4, 0, 1, 3).reshape(BATCH, HIST, NUM_PHONEMES)


def kernel(x, W):
    return _run(x.astype(jnp.int32), W)


# speedup vs baseline: 3.2621x; 1.0004x over previous
import jax
import jax.numpy as jnp
from jax import lax
from jax.experimental import pallas as pl
from jax.experimental.pallas import tpu as pltpu
from jax.experimental.pallas import tpu_sc as plsc

NUM_UNITS = 1000000
NUM_PHONEMES = 32
BATCH = 16384
HIST = 200
HPAD = 208

NW = 32
LANES = 128
BCHUNKS = BATCH // LANES
TC_PER_W = BCHUNKS // NW
DTILES = NUM_PHONEMES // 8


def _gather_kernel(x_hbm, w_hbm, out_hbm, xblock, xblockt, rows, tb,
                   gsem0, gsem1, ssem0, ssem1):
    wid = lax.axis_index("s") * 2 + lax.axis_index("c")
    gsem = (gsem0, gsem1)
    ssem = (ssem0, ssem1)
    jv = lax.iota(jnp.int32, 16)
    bvs = [jv + 16 * v for v in range(8)]

    def transpose_xblock():
        def txb(k, carry):
            dlt = lax.bitwise_and(jv + k, 15)
            for h16 in range(HPAD // 16):
                hj = dlt + 16 * h16
                for v in range(8):
                    val = plsc.load_gather(xblock, [bvs[v], hj])
                    plsc.store_scatter(xblockt, [hj, bvs[v]], val)
            return carry
        lax.fori_loop(0, 16, txb, None)

    def fire_gather(p, h):
        pltpu.async_copy(w_hbm.at[xblockt.at[h]], rows.at[p], gsem[p])

    def drain_gather(p, h):
        pltpu.make_async_copy(w_hbm.at[xblockt.at[h]], rows.at[p],
                              gsem[p]).wait()

    def transpose_chunk(p):
        def tck(k2, carry):
            for u in range(4):
                k = 4 * k2 + u
                dj = lax.bitwise_and(jv + k, 31)
                dt = lax.shift_right_logical(dj, 3)
                dsub = lax.bitwise_and(dj, 7)
                vals = [plsc.load_gather(rows.at[p], [bvs[v], dj])
                        for v in range(8)]
                for v in range(8):
                    plsc.store_scatter(tb.at[p], [dt, dsub, bvs[v]], vals[v])
            return carry
        lax.fori_loop(0, NUM_PHONEMES // 4, tck, None)

    def fire_stores(p, h, tc):
        for tr in range(DTILES):
            pltpu.async_copy(tb.at[p, tr], out_hbm.at[h, tr * BCHUNKS + tc],
                             ssem[p])

    def drain_stores(p, h, tc):
        for tr in range(DTILES):
            pltpu.make_async_copy(tb.at[p, tr],
                                  out_hbm.at[h, tr * BCHUNKS + tc],
                                  ssem[p]).wait()

    def do_block(tc_local, carry):
        tc = wid * TC_PER_W + tc_local
        pltpu.sync_copy(x_hbm.at[pl.ds(tc * LANES, LANES), :],
                        xblock.at[:, pl.ds(0, HIST)])
        transpose_xblock()

        def body(h, p, drain_prev_store, process_prev):
            q = 1 - p
            if drain_prev_store:
                drain_stores(p, h - 2, tc)
            fire_gather(p, h)
            if process_prev:
                drain_gather(q, h - 1)
                transpose_chunk(q)
                fire_stores(q, h - 1, tc)

        body(0, 0, False, False)
        body(1, 1, False, True)

        def loop_body(k, c):
            h = 2 * k
            body(h, 0, True, True)
            body(h + 1, 1, True, True)
            return c

        lax.fori_loop(1, HIST // 2, loop_body, None)

        drain_gather(1, HIST - 1)
        transpose_chunk(1)
        fire_stores(1, HIST - 1, tc)
        drain_stores(0, HIST - 2, tc)
        drain_stores(1, HIST - 1, tc)
        return carry

    lax.fori_loop(0, TC_PER_W, do_block, None)


@jax.jit
def _run(x, w):
    mesh = plsc.VectorSubcoreMesh(core_axis_name="c", subcore_axis_name="s")
    out4 = pl.kernel(
        _gather_kernel,
        out_type=jax.ShapeDtypeStruct((HIST, DTILES * BCHUNKS, 8, LANES),
                                      jnp.float32),
        mesh=mesh,
        scratch_types=[
            pltpu.VMEM((LANES, HPAD), jnp.int32),
            pltpu.VMEM((HPAD, LANES), jnp.int32),
            pltpu.VMEM((2, LANES, NUM_PHONEMES), jnp.float32),
            pltpu.VMEM((2, DTILES, 8, LANES), jnp.float32),
            pltpu.SemaphoreType.DMA,
            pltpu.SemaphoreType.DMA,
            pltpu.SemaphoreType.DMA,
            pltpu.SemaphoreType.DMA,
        ],
        compiler_params=pltpu.CompilerParams(use_tc_tiling_on_sc=False,
                                             needs_layout_passes=False),
    )(x, w)
    out5 = out4.reshape(HIST, DTILES, BCHUNKS, 8, LANES)
    return out5.transpose(2, 4, 0, 1, 3).reshape(BATCH, HIST, NUM_PHONEMES)


def kernel(x, W):
    return _run(x.astype(jnp.int32), W)
